# baseline (speedup 1.0000x reference)
import jax
import jax.numpy as jnp
from jax import lax
from jax.experimental import pallas as pl
from jax.experimental.pallas import tpu as pltpu
from jax.experimental.pallas import tpu_sc as plsc

_NC = 2
_NS = 16
_NW = _NC * _NS
_G = 128
_L = 16
_TR = 8


def kernel(sampled, mask, embed, ind):
    n, d = sampled.shape
    rows_per_w = n // _NW
    chunk = 256
    nbuf = 3
    n_ch = rows_per_w // chunk
    assert rows_per_w * _NW == n and rows_per_w % _G == 0 and d % _L == 0
    assert n_ch * chunk == rows_per_w and n_ch >= nbuf

    mesh = plsc.VectorSubcoreMesh(core_axis_name="c", subcore_axis_name="s")

    def body(samp_hbm, mask_hbm, embed_hbm, ind_hbm, out_hbm,
             mask_v, samp_v, g8_v, idx_v,
             bufs, isems, osems):
        wid = lax.axis_index("s") * _NC + lax.axis_index("c")
        row0 = wid * rows_per_w

        def start_in(c):
            return pltpu.async_copy(
                samp_hbm.at[pl.ds(row0 + c * chunk, chunk)],
                bufs[c % nbuf], isems[c % nbuf])

        def start_out(c):
            return pltpu.async_copy(
                bufs[c % nbuf],
                out_hbm.at[pl.ds(row0 + c * chunk, chunk)],
                osems[c % nbuf])

        ins = {c: start_in(c) for c in range(nbuf)}

        pltpu.sync_copy(mask_hbm.at[pl.ds(row0, rows_per_w)], mask_v)

        def _acc(i, a):
            return a + mask_v[pl.ds(i * _L, _L)]

        acc = lax.fori_loop(0, rows_per_w // _L, _acc,
                            jnp.zeros((_L,), jnp.int32))
        cnt = acc[0]
        for k in range(1, _L):
            cnt = cnt + acc[k]
        all_masked = cnt == rows_per_w

        @pl.when(all_masked)
        def _fast():
            outs = {}
            waited = set()
            for c in range(n_ch):
                ins[c].wait()
                outs[c] = start_out(c)
                nxt = c - 1 + nbuf
                if c >= 1 and nxt < n_ch:
                    outs[c - 1].wait()
                    waited.add(c - 1)
                    ins[nxt] = start_in(nxt)
            for c in range(n_ch):
                if c not in waited:
                    outs[c].wait()

        @pl.when(jnp.logical_not(all_masked))
        def _general():
            for c in range(nbuf):
                ins[c].wait()

            def sub(g, _):
                base = row0 + g * _G
                pltpu.sync_copy(samp_hbm.at[pl.ds(base, _G)], samp_v)
                pltpu.sync_copy(ind_hbm.at[pl.ds(base, _G)], idx_v)

                def take_embed_row(row, idx):
                    def _do():
                        g0 = (idx // _TR) * _TR
                        pltpu.sync_copy(embed_hbm.at[pl.ds(g0, _TR)], g8_v)
                        rr = idx - g0
                        for q in range(d // _L):
                            samp_v[row, pl.ds(q * _L, _L)] = (
                                g8_v[rr, pl.ds(q * _L, _L)])
                    return _do

                def grp(t, _):
                    mvec = mask_v[pl.ds(g * _G + t * _L, _L)]
                    ivec = idx_v[pl.ds(t * _L, _L)]
                    for k in range(_L):
                        pl.when(mvec[k] == 0)(
                            take_embed_row(t * _L + k, ivec[k]))
                    return 0

                lax.fori_loop(0, _G // _L, grp, 0)
                pltpu.sync_copy(samp_v, out_hbm.at[pl.ds(base, _G)])
                return 0

            lax.fori_loop(0, rows_per_w // _G, sub, 0)

    fn = pl.kernel(
        body,
        out_type=jax.ShapeDtypeStruct((n, d), jnp.float32),
        mesh=mesh,
        scratch_types=[
            pltpu.VMEM((rows_per_w,), jnp.int32),
            pltpu.VMEM((_G, d), jnp.float32),
            pltpu.VMEM((_TR, d), jnp.float32),
            pltpu.VMEM((_G,), jnp.int32),
            [pltpu.VMEM((chunk, d), jnp.float32) for _ in range(nbuf)],
            [pltpu.SemaphoreType.DMA for _ in range(nbuf)],
            [pltpu.SemaphoreType.DMA for _ in range(nbuf)],
        ],
    )
    return fn(sampled, mask.astype(jnp.int32), embed, ind)


# speedup vs baseline: 1.0017x; 1.0017x over previous
import jax
import jax.numpy as jnp
from jax import lax
from jax.experimental import pallas as pl
from jax.experimental.pallas import tpu as pltpu
from jax.experimental.pallas import tpu_sc as plsc

_NC = 2
_NS = 16
_NW = _NC * _NS
_G = 128
_L = 16
_TR = 8


def kernel(sampled, mask, embed, ind):
    n, d = sampled.shape
    rows_per_w = n // _NW
    chunk = 128
    nbuf = 4
    n_ch = rows_per_w // chunk
    assert rows_per_w * _NW == n and rows_per_w % _G == 0 and d % _L == 0
    assert n_ch * chunk == rows_per_w and n_ch >= nbuf

    mesh = plsc.VectorSubcoreMesh(core_axis_name="c", subcore_axis_name="s")

    def body(samp_hbm, mask_hbm, embed_hbm, ind_hbm, out_hbm,
             mask_v, samp_v, g8_v, idx_v,
             bufs, isems, osems):
        wid = lax.axis_index("s") * _NC + lax.axis_index("c")
        row0 = wid * rows_per_w

        def start_in(c):
            return pltpu.async_copy(
                samp_hbm.at[pl.ds(row0 + c * chunk, chunk)],
                bufs[c % nbuf], isems[c % nbuf])

        def start_out(c):
            return pltpu.async_copy(
                bufs[c % nbuf],
                out_hbm.at[pl.ds(row0 + c * chunk, chunk)],
                osems[c % nbuf])

        ins = {c: start_in(c) for c in range(nbuf)}

        pltpu.sync_copy(mask_hbm.at[pl.ds(row0, rows_per_w)], mask_v)

        def _acc(i, a):
            return a + mask_v[pl.ds(i * _L, _L)]

        acc = lax.fori_loop(0, rows_per_w // _L, _acc,
                            jnp.zeros((_L,), jnp.int32))
        cnt = acc[0]
        for k in range(1, _L):
            cnt = cnt + acc[k]
        all_masked = cnt == rows_per_w

        @pl.when(all_masked)
        def _fast():
            outs = {}
            waited = set()
            for c in range(n_ch):
                ins[c].wait()
                outs[c] = start_out(c)
                nxt = c - 1 + nbuf
                if c >= 1 and nxt < n_ch:
                    outs[c - 1].wait()
                    waited.add(c - 1)
                    ins[nxt] = start_in(nxt)
            for c in range(n_ch):
                if c not in waited:
                    outs[c].wait()

        @pl.when(jnp.logical_not(all_masked))
        def _general():
            for c in range(nbuf):
                ins[c].wait()

            def sub(g, _):
                base = row0 + g * _G
                pltpu.sync_copy(samp_hbm.at[pl.ds(base, _G)], samp_v)
                pltpu.sync_copy(ind_hbm.at[pl.ds(base, _G)], idx_v)

                def take_embed_row(row, idx):
                    def _do():
                        g0 = (idx // _TR) * _TR
                        pltpu.sync_copy(embed_hbm.at[pl.ds(g0, _TR)], g8_v)
                        rr = idx - g0
                        for q in range(d // _L):
                            samp_v[row, pl.ds(q * _L, _L)] = (
                                g8_v[rr, pl.ds(q * _L, _L)])
                    return _do

                def grp(t, _):
                    mvec = mask_v[pl.ds(g * _G + t * _L, _L)]
                    ivec = idx_v[pl.ds(t * _L, _L)]
                    for k in range(_L):
                        pl.when(mvec[k] == 0)(
                            take_embed_row(t * _L + k, ivec[k]))
                    return 0

                lax.fori_loop(0, _G // _L, grp, 0)
                pltpu.sync_copy(samp_v, out_hbm.at[pl.ds(base, _G)])
                return 0

            lax.fori_loop(0, rows_per_w // _G, sub, 0)

    fn = pl.kernel(
        body,
        out_type=jax.ShapeDtypeStruct((n, d), jnp.float32),
        mesh=mesh,
        scratch_types=[
            pltpu.VMEM((rows_per_w,), jnp.int32),
            pltpu.VMEM((_G, d), jnp.float32),
            pltpu.VMEM((_TR, d), jnp.float32),
            pltpu.VMEM((_G,), jnp.int32),
            [pltpu.VMEM((chunk, d), jnp.float32) for _ in range(nbuf)],
            [pltpu.SemaphoreType.DMA for _ in range(nbuf)],
            [pltpu.SemaphoreType.DMA for _ in range(nbuf)],
        ],
    )
    return fn(sampled, mask.astype(jnp.int32), embed, ind)


# speedup vs baseline: 1.0102x; 1.0085x over previous
import jax
import jax.numpy as jnp
from jax import lax
from jax.experimental import pallas as pl
from jax.experimental.pallas import tpu as pltpu
from jax.experimental.pallas import tpu_sc as plsc

_NC = 2
_NS = 16
_NW = _NC * _NS
_G = 128
_L = 16
_TR = 8


def kernel(sampled, mask, embed, ind):
    n, d = sampled.shape
    rows_per_w = n // _NW
    chunk = 128
    nbuf = 6
    n_ch = rows_per_w // chunk
    assert rows_per_w * _NW == n and rows_per_w % _G == 0 and d % _L == 0
    assert n_ch * chunk == rows_per_w and n_ch >= nbuf

    mesh = plsc.VectorSubcoreMesh(core_axis_name="c", subcore_axis_name="s")

    def body(samp_hbm, mask_hbm, embed_hbm, ind_hbm, out_hbm,
             mask_v, samp_v, g8_v, idx_v,
             bufs, isems, osems):
        wid = lax.axis_index("s") * _NC + lax.axis_index("c")
        row0 = wid * rows_per_w

        def start_in(c):
            return pltpu.async_copy(
                samp_hbm.at[pl.ds(row0 + c * chunk, chunk)],
                bufs[c % nbuf], isems[c % nbuf])

        def start_out(c):
            return pltpu.async_copy(
                bufs[c % nbuf],
                out_hbm.at[pl.ds(row0 + c * chunk, chunk)],
                osems[c % nbuf])

        ins = {c: start_in(c) for c in range(nbuf)}

        pltpu.sync_copy(mask_hbm.at[pl.ds(row0, rows_per_w)], mask_v)

        def _acc(i, a):
            return a + mask_v[pl.ds(i * _L, _L)]

        acc = lax.fori_loop(0, rows_per_w // _L, _acc,
                            jnp.zeros((_L,), jnp.int32))
        cnt = acc[0]
        for k in range(1, _L):
            cnt = cnt + acc[k]
        all_masked = cnt == rows_per_w

        @pl.when(all_masked)
        def _fast():
            outs = {}
            waited = set()
            for c in range(n_ch):
                ins[c].wait()
                outs[c] = start_out(c)
                nxt = c - 1 + nbuf
                if c >= 1 and nxt < n_ch:
                    outs[c - 1].wait()
                    waited.add(c - 1)
                    ins[nxt] = start_in(nxt)
            for c in range(n_ch):
                if c not in waited:
                    outs[c].wait()

        @pl.when(jnp.logical_not(all_masked))
        def _general():
            for c in range(nbuf):
                ins[c].wait()

            def sub(g, _):
                base = row0 + g * _G
                pltpu.sync_copy(samp_hbm.at[pl.ds(base, _G)], samp_v)
                pltpu.sync_copy(ind_hbm.at[pl.ds(base, _G)], idx_v)

                def take_embed_row(row, idx):
                    def _do():
                        g0 = (idx // _TR) * _TR
                        pltpu.sync_copy(embed_hbm.at[pl.ds(g0, _TR)], g8_v)
                        rr = idx - g0
                        for q in range(d // _L):
                            samp_v[row, pl.ds(q * _L, _L)] = (
                                g8_v[rr, pl.ds(q * _L, _L)])
                    return _do

                def grp(t, _):
                    mvec = mask_v[pl.ds(g * _G + t * _L, _L)]
                    ivec = idx_v[pl.ds(t * _L, _L)]
                    for k in range(_L):
                        pl.when(mvec[k] == 0)(
                            take_embed_row(t * _L + k, ivec[k]))
                    return 0

                lax.fori_loop(0, _G // _L, grp, 0)
                pltpu.sync_copy(samp_v, out_hbm.at[pl.ds(base, _G)])
                return 0

            lax.fori_loop(0, rows_per_w // _G, sub, 0)

    fn = pl.kernel(
        body,
        out_type=jax.ShapeDtypeStruct((n, d), jnp.float32),
        mesh=mesh,
        scratch_types=[
            pltpu.VMEM((rows_per_w,), jnp.int32),
            pltpu.VMEM((_G, d), jnp.float32),
            pltpu.VMEM((_TR, d), jnp.float32),
            pltpu.VMEM((_G,), jnp.int32),
            [pltpu.VMEM((chunk, d), jnp.float32) for _ in range(nbuf)],
            [pltpu.SemaphoreType.DMA for _ in range(nbuf)],
            [pltpu.SemaphoreType.DMA for _ in range(nbuf)],
        ],
    )
    return fn(sampled, mask.astype(jnp.int32), embed, ind)
